# Initial kernel scaffold; baseline (speedup 1.0000x reference)
import jax
import jax.numpy as jnp
from jax.experimental import pallas as pl


def kernel(x, Wc):
    raise NotImplementedError("write your pallas kernel here")


# speedup vs baseline: 1.1143x; 1.1143x over previous
import jax
import jax.numpy as jnp
from jax.experimental import pallas as pl
from jax.experimental.pallas import tpu as pltpu

_N, _C, _H, _W = 64, 768, 24, 24
_HW = _H * _W
_TOPK = 128
_K = 3


def _body(x_ref, w_ref, selT_ref, st_ref, out_ref):
    xb = x_ref[0]
    w9 = w_ref[...]

    s = jax.lax.dot_general(w9, xb, (((1,), (0,)), ((), ())),
                            preferred_element_type=jnp.float32)

    hh = jax.lax.broadcasted_iota(jnp.int32, (1, _HW), 1) // _W
    ww = jax.lax.broadcasted_iota(jnp.int32, (1, _HW), 1) % _W
    t = jnp.zeros((1, _HW), jnp.float32)
    for kh in range(_K):
        for kw in range(_K):
            k = kh * _K + kw
            dh, dw = kh - 1, kw - 1
            off = dh * _W + dw
            sk = s[k:k + 1, :]
            if off > 0:
                shifted = jnp.concatenate(
                    [sk[:, off:], jnp.zeros((1, off), jnp.float32)], axis=1)
            elif off < 0:
                shifted = jnp.concatenate(
                    [jnp.zeros((1, -off), jnp.float32), sk[:, :off]], axis=1)
            else:
                shifted = sk
            valid = ((hh + dh >= 0) & (hh + dh < _H)
                     & (ww + dw >= 0) & (ww + dw < _W))
            t = t + jnp.where(valid, shifted, 0.0)

    te = jnp.exp(t)
    tn = te / jnp.sum(te)

    vcol = jnp.transpose(jnp.broadcast_to(tn, (8, _HW)))[:, 0:1]

    ri = jax.lax.broadcasted_iota(jnp.int32, (_HW, _HW), 0)
    ci = jax.lax.broadcasted_iota(jnp.int32, (_HW, _HW), 1)
    ltb = ri < ci
    cnt = jnp.where((vcol > tn) | ((vcol == tn) & ltb), 1.0, 0.0)
    rank = jnp.sum(cnt, axis=0, keepdims=True)
    maskf = jnp.where(rank < float(_TOPK), 1.0, 0.0)

    st = (maskf - tn) + tn
    st_ref[0] = st
    outv = xb * st
    out_ref[0] = outv

    ltf = jnp.where(ltb, 1.0, 0.0)
    pex = jax.lax.dot_general(maskf, ltf, (((1,), (0,)), ((), ())),
                              preferred_element_type=jnp.float32)

    mp = jnp.concatenate([jnp.broadcast_to(maskf, (4, _HW)),
                          jnp.broadcast_to(pex, (4, _HW))], axis=0)
    mpT = jnp.transpose(mp)
    mcol = mpT[:, 0:1]
    pcol = mpT[:, 4:5]

    slot = jax.lax.broadcasted_iota(
        jnp.int32, (1, _TOPK), 1).astype(jnp.float32)
    G = jnp.where((mcol > 0.5) & (pcol == slot), 1.0, 0.0)
    selT = jax.lax.dot_general(G, outv, (((0,), (1,)), ((), ())),
                               preferred_element_type=jnp.float32)
    selT_ref[0] = selT


_GRID_SPEC = dict(
    grid=(_N,),
    in_specs=[
        pl.BlockSpec((1, _C, _HW), lambda i: (i, 0, 0)),
        pl.BlockSpec((16, _C), lambda i: (0, 0)),
    ],
    out_specs=[
        pl.BlockSpec((1, _TOPK, _C), lambda i: (i, 0, 0)),
        pl.BlockSpec((1, 1, _HW), lambda i: (i, 0, 0)),
        pl.BlockSpec((1, _C, _HW), lambda i: (i, 0, 0)),
    ],
)

_OUT_SHAPES = [
    jax.ShapeDtypeStruct((_N, _TOPK, _C), jnp.float32),
    jax.ShapeDtypeStruct((_N, 1, _HW), jnp.float32),
    jax.ShapeDtypeStruct((_N, _C, _HW), jnp.float32),
]


def _run(x3, w16, interpret=False):
    return pl.pallas_call(
        _body,
        out_shape=_OUT_SHAPES,
        compiler_params=pltpu.CompilerParams(
            dimension_semantics=("parallel",)),
        interpret=interpret,
        **_GRID_SPEC,
    )(x3, w16)


def kernel(x, Wc):
    n, c, h, w = x.shape
    x3 = x.reshape(n, c, h * w)
    w9 = jnp.transpose(Wc[0], (1, 2, 0)).reshape(_K * _K, c)
    w16 = jnp.concatenate(
        [w9, jnp.zeros((16 - _K * _K, c), jnp.float32)], axis=0)
    selT, st3, out3 = _run(x3, w16)
    st_mask = st3.reshape(n, 1, h, w)
    out = out3.reshape(n, c, h, w)
    concat_out = jnp.concatenate(
        [selT.reshape(n, _TOPK * c), st3.reshape(n, h * w)], axis=1)
    return (concat_out, st_mask, out)


# speedup vs baseline: 1.1891x; 1.0672x over previous
import jax
import jax.numpy as jnp
from jax.experimental import pallas as pl
from jax.experimental.pallas import tpu as pltpu

_N, _C, _H, _W = 64, 768, 24, 24
_HW = _H * _W
_TOPK = 128
_K = 3


def _body(x_ref, w_ref, cat_ref, st_ref, out_ref):
    xb = x_ref[0]
    w9 = w_ref[...]

    s = jax.lax.dot_general(w9, xb, (((1,), (0,)), ((), ())),
                            preferred_element_type=jnp.float32)

    hh = jax.lax.broadcasted_iota(jnp.int32, (1, _HW), 1) // _W
    ww = jax.lax.broadcasted_iota(jnp.int32, (1, _HW), 1) % _W
    t = jnp.zeros((1, _HW), jnp.float32)
    for kh in range(_K):
        for kw in range(_K):
            k = kh * _K + kw
            dh, dw = kh - 1, kw - 1
            off = dh * _W + dw
            sk = s[k:k + 1, :]
            if off > 0:
                shifted = jnp.concatenate(
                    [sk[:, off:], jnp.zeros((1, off), jnp.float32)], axis=1)
            elif off < 0:
                shifted = jnp.concatenate(
                    [jnp.zeros((1, -off), jnp.float32), sk[:, :off]], axis=1)
            else:
                shifted = sk
            valid = ((hh + dh >= 0) & (hh + dh < _H)
                     & (ww + dw >= 0) & (ww + dw < _W))
            t = t + jnp.where(valid, shifted, 0.0)

    te = jnp.exp(t)
    tn = te / jnp.sum(te)

    vcol = jnp.transpose(jnp.broadcast_to(tn, (8, _HW)))[:, 0:1]

    ri = jax.lax.broadcasted_iota(jnp.int32, (_HW, _HW), 0)
    ci = jax.lax.broadcasted_iota(jnp.int32, (_HW, _HW), 1)
    ltb = ri < ci
    cnt = jnp.where((vcol > tn) | ((vcol == tn) & ltb), 1.0, 0.0)
    rank = jnp.sum(cnt, axis=0, keepdims=True)
    maskf = jnp.where(rank < float(_TOPK), 1.0, 0.0)

    st = (maskf - tn) + tn
    st_ref[0] = st
    outv = xb * st
    out_ref[0] = outv

    ltf = jnp.where(ltb, 1.0, 0.0)
    pex = jax.lax.dot_general(maskf, ltf, (((1,), (0,)), ((), ())),
                              preferred_element_type=jnp.float32)

    mp = jnp.concatenate([jnp.broadcast_to(maskf, (4, _HW)),
                          jnp.broadcast_to(pex, (4, _HW))], axis=0)
    mpT = jnp.transpose(mp)
    mcol = mpT[:, 0:1]
    pcol = mpT[:, 4:5]

    slot = jax.lax.broadcasted_iota(
        jnp.int32, (1, _TOPK), 1).astype(jnp.float32)
    G = jnp.where((mcol > 0.5) & (pcol == slot), 1.0, 0.0)
    selT = jax.lax.dot_general(G, outv, (((0,), (1,)), ((), ())),
                               preferred_element_type=jnp.float32)
    for p in range(_TOPK):
        cat_ref[0, 0, p * _C:(p + 1) * _C] = selT[p, :]
    cat_ref[0, 0, _TOPK * _C:] = st[0, :]


_GRID_SPEC = dict(
    grid=(_N,),
    in_specs=[
        pl.BlockSpec((1, _C, _HW), lambda i: (i, 0, 0)),
        pl.BlockSpec((16, _C), lambda i: (0, 0)),
    ],
    out_specs=[
        pl.BlockSpec((1, 1, _TOPK * _C + _HW), lambda i: (i, 0, 0)),
        pl.BlockSpec((1, 1, _HW), lambda i: (i, 0, 0)),
        pl.BlockSpec((1, _C, _HW), lambda i: (i, 0, 0)),
    ],
)

_OUT_SHAPES = [
    jax.ShapeDtypeStruct((_N, 1, _TOPK * _C + _HW), jnp.float32),
    jax.ShapeDtypeStruct((_N, 1, _HW), jnp.float32),
    jax.ShapeDtypeStruct((_N, _C, _HW), jnp.float32),
]


def _run(x3, w16, interpret=False):
    return pl.pallas_call(
        _body,
        out_shape=_OUT_SHAPES,
        compiler_params=pltpu.CompilerParams(
            dimension_semantics=("parallel",)),
        interpret=interpret,
        **_GRID_SPEC,
    )(x3, w16)


def kernel(x, Wc):
    n, c, h, w = x.shape
    x3 = x.reshape(n, c, h * w)
    w9 = jnp.transpose(Wc[0], (1, 2, 0)).reshape(_K * _K, c)
    w16 = jnp.concatenate(
        [w9, jnp.zeros((16 - _K * _K, c), jnp.float32)], axis=0)
    cat3, st3, out3 = _run(x3, w16)
    st_mask = st3.reshape(n, 1, h, w)
    out = out3.reshape(n, c, h, w)
    concat_out = cat3.reshape(n, _TOPK * c + h * w)
    return (concat_out, st_mask, out)
